# baseline (speedup 1.0000x reference)
import functools

import jax
import jax.numpy as jnp
from jax import lax
from jax.experimental import pallas as pl
from jax.experimental.pallas import tpu as pltpu
from jax.experimental.pallas import tpu_sc as plsc

DIM = 64
L = 20
SAMPLE_NUM = 12
ALPHA = 0.2


def _sc_gather(emb2, abc_row, pair_idx, adj_flat, n_abc, n_pair):
    info = plsc.get_sparse_core_info()
    NC, NS = info.num_cores, info.num_subcores
    NW = NC * NS

    per_a = n_abc // NW
    per_p = n_pair // NW
    C = 128
    n_ca = per_a // C
    n_cp = per_p // C

    mesh = plsc.VectorSubcoreMesh(core_axis_name="c", subcore_axis_name="s")

    @functools.partial(
        pl.kernel,
        mesh=mesh,
        out_type=[
            jax.ShapeDtypeStruct((n_abc, 2 * DIM), jnp.float32),
            jax.ShapeDtypeStruct((n_pair, 2 * DIM), jnp.float32),
            jax.ShapeDtypeStruct((n_pair,), jnp.int32),
        ],
        scratch_types=[
            pltpu.VMEM((max(per_a, per_p),), jnp.int32),
            pltpu.VMEM((C,), jnp.int32),
            pltpu.VMEM((C,), jnp.int32),
            pltpu.VMEM((C, 2 * DIM), jnp.float32),
            pltpu.SemaphoreType.DMA,
        ],
    )
    def k(emb_hbm, abc_hbm, pair_hbm, adjflat_hbm,
          out1_hbm, out2_hbm, out2i_hbm,
          ibuf, pbuf, rbuf, rows, sem):
        wid = lax.axis_index("s") * NC + lax.axis_index("c")

        base_a = wid * per_a
        pltpu.sync_copy(abc_hbm.at[pl.ds(base_a, per_a)],
                        ibuf.at[pl.ds(0, per_a)])

        def pa_body(c, carry):
            pltpu.async_copy(
                emb_hbm.at[ibuf.at[pl.ds(c * C, C)]], rows, sem).wait()
            pltpu.sync_copy(rows, out1_hbm.at[pl.ds(base_a + c * C, C)])
            return carry

        lax.fori_loop(0, n_ca, pa_body, 0, unroll=False)

        base_p = wid * per_p
        pltpu.sync_copy(pair_hbm.at[pl.ds(base_p, per_p)], ibuf)

        def pb_body(c, carry):
            pltpu.async_copy(
                adjflat_hbm.at[ibuf.at[pl.ds(c * C, C)]], pbuf, sem).wait()
            pltpu.sync_copy(pbuf, out2i_hbm.at[pl.ds(base_p + c * C, C)])

            def shift(s, carry2):
                rbuf[pl.ds(s * 16, 16)] = lax.shift_right_logical(
                    pbuf[pl.ds(s * 16, 16)], 1)
                return carry2

            lax.fori_loop(0, C // 16, shift, 0, unroll=False)
            pltpu.async_copy(emb_hbm.at[rbuf], rows, sem).wait()
            pltpu.sync_copy(rows, out2_hbm.at[pl.ds(base_p + c * C, C)])
            return carry

        lax.fori_loop(0, n_cp, pb_body, 0, unroll=False)

    return k(emb2, abc_row, pair_idx, adj_flat)


def _leaky(x):
    return jnp.where(x >= 0, x, ALPHA * x)


def _bmm(x, y, dims):
    return lax.dot_general(x, y, dims, preferred_element_type=jnp.float32)


def _sel(blk, par):
    return jnp.where(par[..., None] > 0.5, blk[..., DIM:], blk[..., :DIM])


def _tc_body(adj_ref, mask_ref, a_ref, g0w1_ref, g0w2_ref, g0w3_ref,
             g1w1_ref, g1w2_ref, g1w3_ref, h_ref, hp_ref, item_ref, ip_ref,
             first_ref, fp_ref, second_ref, sp_ref, out_ref):
    bb = h_ref.shape[0]
    h = _sel(h_ref[...], hp_ref[...])
    amat = a_ref[...]

    adj = adj_ref[...]
    e = []
    for kk in range(4):
        ha = h * amat[:, kk][None, None, :]
        ek = _bmm(ha, h, (((2,), (2,)), ((0,), (0,))))
        e.append(_leaky(ek))
    alpha = jnp.full_like(e[0], -9e15)
    alpha = jnp.where(adj == 1, e[0], alpha)
    alpha = jnp.where(adj == 2, e[1], alpha)
    alpha = jnp.where(adj == 3, e[2], alpha)
    alpha = jnp.where(adj == 4, e[3], alpha)
    m = jnp.max(alpha, axis=-1, keepdims=True)
    ex = jnp.exp(alpha - m)
    alpha = ex / jnp.sum(ex, axis=-1, keepdims=True)
    h_local = _bmm(alpha, h, (((2,), (1,)), ((0,), (0,))))

    mask = mask_ref[...]
    item_emb = _sel(item_ref[...], ip_ref[...]) * mask[..., None]
    s = jnp.sum(item_emb, axis=1) / jnp.sum(mask, axis=-1)[..., None]

    first = _sel(first_ref[...], fp_ref[...])
    second = _sel(second_ref[...], sp_ref[...])

    def gagg(selfv, nb, n, kk, w1, w2, w3):
        x = nb * s[:, None, None, :]
        a = _leaky(_bmm(x.reshape(bb * n * kk, DIM), w1, (((1,), (0,)), ((), ()))))
        logits = _bmm(a, w2, (((1,), (0,)), ((), ()))).reshape(bb, n, kk)
        mx = jnp.max(logits, axis=-1, keepdims=True)
        exl = jnp.exp(logits - mx)
        attn = exl / jnp.sum(exl, axis=-1, keepdims=True)
        agg = jnp.sum(attn[..., None] * nb, axis=2)
        cat = jnp.concatenate([selfv, agg], axis=-1)
        o = _bmm(cat.reshape(bb * n, 2 * DIM), w3, (((1,), (0,)), ((), ())))
        return jnp.maximum(o, 0.0).reshape(bb, n, DIM)

    g0w1, g0w2, g0w3 = g0w1_ref[...], g0w2_ref[...], g0w3_ref[...]
    g1w1, g1w2, g1w3 = g1w1_ref[...], g1w2_ref[...], g1w3_ref[...]

    n1 = L * SAMPLE_NUM
    hop0a = gagg(h, first.reshape(bb, L, SAMPLE_NUM, DIM),
                 L, SAMPLE_NUM, g0w1, g0w2, g0w3)
    hop0b = gagg(first, second.reshape(bb, n1, 2, DIM),
                 n1, 2, g0w1, g0w2, g0w3)
    hop1 = gagg(hop0a, hop0b.reshape(bb, L, SAMPLE_NUM, DIM),
                L, SAMPLE_NUM, g1w1, g1w2, g1w3)

    out_ref[...] = h_local + hop1


def _tc_compute(adj, mask_item, amat, g0w1, g0w2, g0w3, g1w1, g1w2, g1w3,
                h_rows, h_par, item_rows, item_par, first_rows, first_par,
                second_rows, second_par, bb):
    b = adj.shape[0]
    grid = (b // bb,)

    def blk(*shape):
        return pl.BlockSpec((bb,) + shape, lambda i: (i,) + (0,) * len(shape))

    def full(arr):
        return pl.BlockSpec(arr.shape, lambda i: (0,) * arr.ndim)

    return pl.pallas_call(
        _tc_body,
        grid=grid,
        in_specs=[
            blk(L, L),
            blk(L),
            full(amat), full(g0w1), full(g0w2), full(g0w3),
            full(g1w1), full(g1w2), full(g1w3),
            blk(L, 2 * DIM),
            blk(L),
            blk(L, 2 * DIM),
            blk(L),
            blk(L * SAMPLE_NUM, 2 * DIM),
            blk(L * SAMPLE_NUM),
            blk(2 * L * SAMPLE_NUM, 2 * DIM),
            blk(2 * L * SAMPLE_NUM),
        ],
        out_specs=blk(L, DIM),
        out_shape=jax.ShapeDtypeStruct((b, L, DIM), jnp.float32),
    )(adj, mask_item, amat, g0w1, g0w2, g0w3, g1w1, g1w2, g1w3,
      h_rows, h_par, item_rows, item_par, first_rows, first_par,
      second_rows, second_par)


def kernel(inputs, adj, mask_item, item, first_adj, adj_all, emb,
           la_a0, la_a1, la_a2, la_a3,
           g0_w1, g0_w2, g0_w3, g1_w1, g1_w2, g1_w3):
    b, seq = inputs.shape
    n_in = b * seq
    n_first = b * seq * SAMPLE_NUM
    n_abc = 2 * n_in + n_first
    n_pair = 2 * n_first

    idx_abc = jnp.concatenate([
        inputs.reshape(-1), item.reshape(-1), first_adj.reshape(-1)
    ]).astype(jnp.int32)
    abc_row = jnp.right_shift(idx_abc, 1)
    abc_par = jnp.bitwise_and(idx_abc, 1).astype(jnp.float32)
    ff = first_adj.reshape(-1).astype(jnp.int32)
    pair_idx = jnp.stack([ff * 2, ff * 2 + 1], axis=-1).reshape(-1)
    adj_flat = adj_all.reshape(-1).astype(jnp.int32)

    emb2 = emb.reshape(-1, 2 * DIM)
    out1, out2, out2i = _sc_gather(emb2, abc_row, pair_idx, adj_flat,
                                   n_abc=n_abc, n_pair=n_pair)

    h_rows = out1[:n_in].reshape(b, seq, 2 * DIM)
    item_rows = out1[n_in:2 * n_in].reshape(b, seq, 2 * DIM)
    first_rows = out1[2 * n_in:].reshape(b, seq * SAMPLE_NUM, 2 * DIM)
    second_rows = out2.reshape(b, 2 * seq * SAMPLE_NUM, 2 * DIM)

    h_par = abc_par[:n_in].reshape(b, seq)
    item_par = abc_par[n_in:2 * n_in].reshape(b, seq)
    first_par = abc_par[2 * n_in:].reshape(b, seq * SAMPLE_NUM)
    second_par = jnp.bitwise_and(out2i, 1).astype(jnp.float32).reshape(
        b, 2 * seq * SAMPLE_NUM)

    amat = jnp.concatenate([la_a0, la_a1, la_a2, la_a3], axis=1)

    out = _tc_compute(adj.astype(jnp.int32), mask_item, amat,
                      g0_w1, g0_w2, g0_w3, g1_w1, g1_w2, g1_w3,
                      h_rows, h_par, item_rows, item_par,
                      first_rows, first_par, second_rows, second_par, bb=8)
    return out


# speedup vs baseline: 1.5562x; 1.5562x over previous
import functools

import jax
import jax.numpy as jnp
from jax import lax
from jax.experimental import pallas as pl
from jax.experimental.pallas import tpu as pltpu
from jax.experimental.pallas import tpu_sc as plsc

DIM = 64
L = 20
SAMPLE_NUM = 12
ALPHA = 0.2


def _sc_gather(emb2, abc_row, pair_idx, adj_flat, n_abc, n_pair):
    info = plsc.get_sparse_core_info()
    NC, NS = info.num_cores, info.num_subcores
    NW = NC * NS

    per_a = n_abc // NW
    per_p = n_pair // NW
    C = 128
    n_ca = per_a // C
    n_cp = per_p // C

    mesh = plsc.VectorSubcoreMesh(core_axis_name="c", subcore_axis_name="s")

    @functools.partial(
        pl.kernel,
        mesh=mesh,
        out_type=[
            jax.ShapeDtypeStruct((n_abc, 2 * DIM), jnp.float32),
            jax.ShapeDtypeStruct((n_pair, 2 * DIM), jnp.float32),
            jax.ShapeDtypeStruct((n_pair,), jnp.int32),
        ],
        scratch_types=[
            pltpu.VMEM((max(per_a, per_p),), jnp.int32),
            pltpu.VMEM((C,), jnp.int32),
            pltpu.VMEM((C,), jnp.int32),
            pltpu.VMEM((C, 2 * DIM), jnp.float32),
            pltpu.SemaphoreType.DMA,
        ],
    )
    def k(emb_hbm, abc_hbm, pair_hbm, adjflat_hbm,
          out1_hbm, out2_hbm, out2i_hbm,
          ibuf, pbuf, rbuf, rows, sem):
        wid = lax.axis_index("s") * NC + lax.axis_index("c")

        base_a = wid * per_a
        pltpu.sync_copy(abc_hbm.at[pl.ds(base_a, per_a)],
                        ibuf.at[pl.ds(0, per_a)])

        def pa_body(c, carry):
            pltpu.async_copy(
                emb_hbm.at[ibuf.at[pl.ds(c * C, C)]], rows, sem).wait()
            pltpu.sync_copy(rows, out1_hbm.at[pl.ds(base_a + c * C, C)])
            return carry

        lax.fori_loop(0, n_ca, pa_body, 0, unroll=False)

        base_p = wid * per_p
        pltpu.sync_copy(pair_hbm.at[pl.ds(base_p, per_p)], ibuf)

        def pb_body(c, carry):
            pltpu.async_copy(
                adjflat_hbm.at[ibuf.at[pl.ds(c * C, C)]], pbuf, sem).wait()
            pltpu.sync_copy(pbuf, out2i_hbm.at[pl.ds(base_p + c * C, C)])

            def shift(s, carry2):
                rbuf[pl.ds(s * 16, 16)] = lax.shift_right_logical(
                    pbuf[pl.ds(s * 16, 16)], 1)
                return carry2

            lax.fori_loop(0, C // 16, shift, 0, unroll=False)
            pltpu.async_copy(emb_hbm.at[rbuf], rows, sem).wait()
            pltpu.sync_copy(rows, out2_hbm.at[pl.ds(base_p + c * C, C)])
            return carry

        lax.fori_loop(0, n_cp, pb_body, 0, unroll=False)

    return k(emb2, abc_row, pair_idx, adj_flat)


def _leaky(x):
    return jnp.where(x >= 0, x, ALPHA * x)


def _bmm(x, y, dims):
    return lax.dot_general(x, y, dims, preferred_element_type=jnp.float32)


def _sel(blk, par):
    return jnp.where(par[..., None] > 0.5, blk[..., DIM:], blk[..., :DIM])


def _gagg(selfv, nb, s_m, w1, w2, w3):
    kk, m, _ = nb.shape
    x = nb * s_m[None]
    a = _leaky(_bmm(x.reshape(kk * m, DIM), w1, (((1,), (0,)), ((), ()))))
    logits = _bmm(a, w2, (((1,), (0,)), ((), ()))).reshape(kk, m)
    mx = jnp.max(logits, axis=0, keepdims=True)
    exl = jnp.exp(logits - mx)
    attn = exl / jnp.sum(exl, axis=0, keepdims=True)
    agg = jnp.sum(attn[..., None] * nb, axis=0)
    cat = jnp.concatenate([selfv, agg], axis=-1)
    o = _bmm(cat, w3, (((1,), (0,)), ((), ())))
    return jnp.maximum(o, 0.0)


def _tc_body(emb2_ref, adj_ref, mask_ref, a_ref, g0w1_ref, g0w2_ref,
             g0w3_ref, g1w1_ref, g1w2_ref, g1w3_ref, h_ref, hp_ref,
             item_ref, ip_ref, first_ref, fp_ref, second_ref, sp_ref,
             out_ref):
    del emb2_ref
    bb = h_ref.shape[0]
    m = bb * L
    h = _sel(h_ref[...], hp_ref[...])
    amat = a_ref[...]

    adj = adj_ref[...]
    e = []
    for kk in range(4):
        ha = h * amat[:, kk][None, None, :]
        ek = _bmm(ha, h, (((2,), (2,)), ((0,), (0,))))
        e.append(_leaky(ek))
    alpha = jnp.full_like(e[0], -9e15)
    alpha = jnp.where(adj == 1, e[0], alpha)
    alpha = jnp.where(adj == 2, e[1], alpha)
    alpha = jnp.where(adj == 3, e[2], alpha)
    alpha = jnp.where(adj == 4, e[3], alpha)
    mxl = jnp.max(alpha, axis=-1, keepdims=True)
    ex = jnp.exp(alpha - mxl)
    alpha = ex / jnp.sum(ex, axis=-1, keepdims=True)
    h_local = _bmm(alpha, h, (((2,), (1,)), ((0,), (0,))))

    mask = mask_ref[...]
    item_emb = _sel(item_ref[...], ip_ref[...]) * mask[..., None]
    s = jnp.sum(item_emb, axis=1) / jnp.sum(mask, axis=-1)[..., None]
    s_bl = jnp.broadcast_to(s[:, None, :], (bb, L, DIM)).reshape(m, DIM)
    s_kbl = jnp.broadcast_to(s_bl[None], (SAMPLE_NUM, m, DIM)).reshape(
        SAMPLE_NUM * m, DIM)

    first = _sel(first_ref[...], fp_ref[...])
    second = _sel(second_ref[...], sp_ref[...])

    g0w1, g0w2, g0w3 = g0w1_ref[...], g0w2_ref[...], g0w3_ref[...]
    g1w1, g1w2, g1w3 = g1w1_ref[...], g1w2_ref[...], g1w3_ref[...]

    h_flat = h.reshape(m, DIM)
    first_flat = first.reshape(SAMPLE_NUM * m, DIM)
    hop0a = _gagg(h_flat, first, s_bl, g0w1, g0w2, g0w3)
    hop0b = _gagg(first_flat, second.reshape(2, SAMPLE_NUM * m, DIM),
                  s_kbl, g0w1, g0w2, g0w3)
    hop1 = _gagg(hop0a, hop0b.reshape(SAMPLE_NUM, m, DIM),
                 s_bl, g1w1, g1w2, g1w3)

    out_ref[...] = h_local + hop1.reshape(bb, L, DIM)


def _tc_compute(emb2, adj, mask_item, amat, g0w1, g0w2, g0w3, g1w1, g1w2,
                g1w3, h_rows, h_par, item_rows, item_par, first_rows,
                first_par, second_rows, second_par, bb):
    b = adj.shape[0]
    grid = (b // bb,)
    m = bb * L

    def blk(*shape):
        return pl.BlockSpec((bb,) + shape, lambda i: (i,) + (0,) * len(shape))

    def kblk(*lead):
        nl = len(lead)
        return pl.BlockSpec(lead + (m, 128), lambda i: (0,) * nl + (i, 0))

    def kpar(*lead):
        nl = len(lead)
        return pl.BlockSpec(lead + (m,), lambda i: (0,) * nl + (i,))

    def full(arr):
        return pl.BlockSpec(arr.shape, lambda i: (0,) * arr.ndim)

    return pl.pallas_call(
        _tc_body,
        grid=grid,
        in_specs=[
            pl.BlockSpec((8, 128), lambda i: (0, 0)),
            blk(L, L),
            blk(L),
            full(amat), full(g0w1), full(g0w2), full(g0w3),
            full(g1w1), full(g1w2), full(g1w3),
            blk(L, 2 * DIM),
            blk(L),
            blk(L, 2 * DIM),
            blk(L),
            kblk(SAMPLE_NUM),
            kpar(SAMPLE_NUM),
            kblk(2, SAMPLE_NUM),
            kpar(2, SAMPLE_NUM),
        ],
        out_specs=blk(L, DIM),
        out_shape=jax.ShapeDtypeStruct((b, L, DIM), jnp.float32),
    )(emb2, adj, mask_item, amat, g0w1, g0w2, g0w3, g1w1, g1w2, g1w3,
      h_rows, h_par, item_rows, item_par, first_rows, first_par,
      second_rows, second_par)


def kernel(inputs, adj, mask_item, item, first_adj, adj_all, emb,
           la_a0, la_a1, la_a2, la_a3,
           g0_w1, g0_w2, g0_w3, g1_w1, g1_w2, g1_w3):
    b, seq = inputs.shape
    n_in = b * seq
    n_first = b * seq * SAMPLE_NUM
    n_abc = 2 * n_in + n_first
    n_pair = 2 * n_first

    ff_t = jnp.transpose(first_adj, (2, 0, 1)).reshape(-1).astype(jnp.int32)
    idx_abc = jnp.concatenate([
        inputs.reshape(-1).astype(jnp.int32),
        item.reshape(-1).astype(jnp.int32),
        ff_t,
    ])
    abc_row = jnp.right_shift(idx_abc, 1)
    abc_par = jnp.bitwise_and(idx_abc, 1).astype(jnp.float32)
    pair_idx = jnp.concatenate([ff_t * 2, ff_t * 2 + 1])
    adj_flat = adj_all.reshape(-1).astype(jnp.int32)

    emb2 = emb.reshape(-1, 2 * DIM)
    out1, out2, out2i = _sc_gather(emb2, abc_row, pair_idx, adj_flat,
                                   n_abc=n_abc, n_pair=n_pair)

    h_rows = out1[:n_in].reshape(b, seq, 2 * DIM)
    item_rows = out1[n_in:2 * n_in].reshape(b, seq, 2 * DIM)
    first_rows = out1[2 * n_in:].reshape(SAMPLE_NUM, b * seq, 2 * DIM)
    second_rows = out2.reshape(2, SAMPLE_NUM, b * seq, 2 * DIM)

    h_par = abc_par[:n_in].reshape(b, seq)
    item_par = abc_par[n_in:2 * n_in].reshape(b, seq)
    first_par = abc_par[2 * n_in:].reshape(SAMPLE_NUM, b * seq)
    second_par = jnp.bitwise_and(out2i, 1).astype(jnp.float32).reshape(
        2, SAMPLE_NUM, b * seq)

    amat = jnp.concatenate([la_a0, la_a1, la_a2, la_a3], axis=1)

    out = _tc_compute(emb2, adj.astype(jnp.int32), mask_item, amat,
                      g0_w1, g0_w2, g0_w3, g1_w1, g1_w2, g1_w3,
                      h_rows, h_par, item_rows, item_par,
                      first_rows, first_par, second_rows, second_par, bb=32)
    return out


# speedup vs baseline: 1.5578x; 1.0010x over previous
import functools

import jax
import jax.numpy as jnp
from jax import lax
from jax.experimental import pallas as pl
from jax.experimental.pallas import tpu as pltpu
from jax.experimental.pallas import tpu_sc as plsc

DIM = 64
L = 20
SAMPLE_NUM = 12
ALPHA = 0.2


def _sc_gather(emb2, abc_row, pair_idx, adj_flat, n_abc, n_pair):
    info = plsc.get_sparse_core_info()
    NC, NS = info.num_cores, info.num_subcores
    NW = NC * NS

    per_a = n_abc // NW
    per_p = n_pair // NW
    C = 128
    n_ca = per_a // C
    n_cp = per_p // C

    mesh = plsc.VectorSubcoreMesh(core_axis_name="c", subcore_axis_name="s")

    @functools.partial(
        pl.kernel,
        mesh=mesh,
        compiler_params=pltpu.CompilerParams(use_tc_tiling_on_sc=True),
        out_type=[
            jax.ShapeDtypeStruct((n_abc, 2 * DIM), jnp.float32),
            jax.ShapeDtypeStruct((n_pair, 2 * DIM), jnp.float32),
            jax.ShapeDtypeStruct((n_pair,), jnp.int32),
        ],
        scratch_types=[
            pltpu.VMEM((max(per_a, per_p),), jnp.int32),
            pltpu.VMEM((C,), jnp.int32),
            pltpu.VMEM((C,), jnp.int32),
            pltpu.VMEM((C, 2 * DIM), jnp.float32),
            pltpu.SemaphoreType.DMA,
        ],
    )
    def k(emb_hbm, abc_hbm, pair_hbm, adjflat_hbm,
          out1_hbm, out2_hbm, out2i_hbm,
          ibuf, pbuf, rbuf, rows, sem):
        wid = lax.axis_index("s") * NC + lax.axis_index("c")

        base_a = wid * per_a
        pltpu.sync_copy(abc_hbm.at[pl.ds(base_a, per_a)],
                        ibuf.at[pl.ds(0, per_a)])

        def pa_body(c, carry):
            pltpu.async_copy(
                emb_hbm.at[ibuf.at[pl.ds(c * C, C)]], rows, sem).wait()
            pltpu.sync_copy(rows, out1_hbm.at[pl.ds(base_a + c * C, C)])
            return carry

        lax.fori_loop(0, n_ca, pa_body, 0, unroll=False)

        base_p = wid * per_p
        pltpu.sync_copy(pair_hbm.at[pl.ds(base_p, per_p)], ibuf)

        def pb_body(c, carry):
            pltpu.async_copy(
                adjflat_hbm.at[ibuf.at[pl.ds(c * C, C)]], pbuf, sem).wait()
            pltpu.sync_copy(pbuf, out2i_hbm.at[pl.ds(base_p + c * C, C)])

            def shift(s, carry2):
                rbuf[pl.ds(s * 16, 16)] = lax.shift_right_logical(
                    pbuf[pl.ds(s * 16, 16)], 1)
                return carry2

            lax.fori_loop(0, C // 16, shift, 0, unroll=False)
            pltpu.async_copy(emb_hbm.at[rbuf], rows, sem).wait()
            pltpu.sync_copy(rows, out2_hbm.at[pl.ds(base_p + c * C, C)])
            return carry

        lax.fori_loop(0, n_cp, pb_body, 0, unroll=False)

    return k(emb2, abc_row, pair_idx, adj_flat)


def _leaky(x):
    return jnp.where(x >= 0, x, ALPHA * x)


def _bmm(x, y, dims):
    return lax.dot_general(x, y, dims, preferred_element_type=jnp.float32)


def _sel(blk, par):
    return jnp.where(par[..., None] > 0.5, blk[..., DIM:], blk[..., :DIM])


def _gagg(selfv, nb, s_m, w1, w2, w3):
    kk, m, _ = nb.shape
    x = nb * s_m[None]
    a = _leaky(_bmm(x.reshape(kk * m, DIM), w1, (((1,), (0,)), ((), ()))))
    logits = _bmm(a, w2, (((1,), (0,)), ((), ()))).reshape(kk, m)
    mx = jnp.max(logits, axis=0, keepdims=True)
    exl = jnp.exp(logits - mx)
    attn = exl / jnp.sum(exl, axis=0, keepdims=True)
    agg = jnp.sum(attn[..., None] * nb, axis=0)
    cat = jnp.concatenate([selfv, agg], axis=-1)
    o = _bmm(cat, w3, (((1,), (0,)), ((), ())))
    return jnp.maximum(o, 0.0)


def _tc_body(emb2_ref, adj_ref, mask_ref, a_ref, g0w1_ref, g0w2_ref,
             g0w3_ref, g1w1_ref, g1w2_ref, g1w3_ref, h_ref, hp_ref,
             item_ref, ip_ref, first_ref, fp_ref, second_ref, sp_ref,
             out_ref):
    del emb2_ref
    bb = h_ref.shape[0]
    m = bb * L
    h = _sel(h_ref[...], hp_ref[...])
    amat = a_ref[...]

    adj = adj_ref[...]
    e = []
    for kk in range(4):
        ha = h * amat[:, kk][None, None, :]
        ek = _bmm(ha, h, (((2,), (2,)), ((0,), (0,))))
        e.append(_leaky(ek))
    alpha = jnp.full_like(e[0], -9e15)
    alpha = jnp.where(adj == 1, e[0], alpha)
    alpha = jnp.where(adj == 2, e[1], alpha)
    alpha = jnp.where(adj == 3, e[2], alpha)
    alpha = jnp.where(adj == 4, e[3], alpha)
    mxl = jnp.max(alpha, axis=-1, keepdims=True)
    ex = jnp.exp(alpha - mxl)
    alpha = ex / jnp.sum(ex, axis=-1, keepdims=True)
    h_local = _bmm(alpha, h, (((2,), (1,)), ((0,), (0,))))

    mask = mask_ref[...]
    item_emb = _sel(item_ref[...], ip_ref[...]) * mask[..., None]
    s = jnp.sum(item_emb, axis=1) / jnp.sum(mask, axis=-1)[..., None]
    s_bl = jnp.broadcast_to(s[:, None, :], (bb, L, DIM)).reshape(m, DIM)
    s_kbl = jnp.broadcast_to(s_bl[None], (SAMPLE_NUM, m, DIM)).reshape(
        SAMPLE_NUM * m, DIM)

    first = _sel(first_ref[...], fp_ref[...])
    second = _sel(second_ref[...], sp_ref[...])

    g0w1, g0w2, g0w3 = g0w1_ref[...], g0w2_ref[...], g0w3_ref[...]
    g1w1, g1w2, g1w3 = g1w1_ref[...], g1w2_ref[...], g1w3_ref[...]

    h_flat = h.reshape(m, DIM)
    first_flat = first.reshape(SAMPLE_NUM * m, DIM)
    hop0a = _gagg(h_flat, first, s_bl, g0w1, g0w2, g0w3)
    hop0b = _gagg(first_flat, second.reshape(2, SAMPLE_NUM * m, DIM),
                  s_kbl, g0w1, g0w2, g0w3)
    hop1 = _gagg(hop0a, hop0b.reshape(SAMPLE_NUM, m, DIM),
                 s_bl, g1w1, g1w2, g1w3)

    out_ref[...] = h_local + hop1.reshape(bb, L, DIM)


def _tc_compute(emb2, adj, mask_item, amat, g0w1, g0w2, g0w3, g1w1, g1w2,
                g1w3, h_rows, h_par, item_rows, item_par, first_rows,
                first_par, second_rows, second_par, bb):
    b = adj.shape[0]
    grid = (b // bb,)
    m = bb * L

    def blk(*shape):
        return pl.BlockSpec((bb,) + shape, lambda i: (i,) + (0,) * len(shape))

    def kblk(*lead):
        nl = len(lead)
        return pl.BlockSpec(lead + (m, 128), lambda i: (0,) * nl + (i, 0))

    def kpar(*lead):
        nl = len(lead)
        return pl.BlockSpec(lead + (m,), lambda i: (0,) * nl + (i,))

    def full(arr):
        return pl.BlockSpec(arr.shape, lambda i: (0,) * arr.ndim)

    return pl.pallas_call(
        _tc_body,
        grid=grid,
        in_specs=[
            pl.BlockSpec((8, 128), lambda i: (0, 0)),
            blk(L, L),
            blk(L),
            full(amat), full(g0w1), full(g0w2), full(g0w3),
            full(g1w1), full(g1w2), full(g1w3),
            blk(L, 2 * DIM),
            blk(L),
            blk(L, 2 * DIM),
            blk(L),
            kblk(SAMPLE_NUM),
            kpar(SAMPLE_NUM),
            kblk(2, SAMPLE_NUM),
            kpar(2, SAMPLE_NUM),
        ],
        out_specs=blk(L, DIM),
        out_shape=jax.ShapeDtypeStruct((b, L, DIM), jnp.float32),
    )(emb2, adj, mask_item, amat, g0w1, g0w2, g0w3, g1w1, g1w2, g1w3,
      h_rows, h_par, item_rows, item_par, first_rows, first_par,
      second_rows, second_par)


def kernel(inputs, adj, mask_item, item, first_adj, adj_all, emb,
           la_a0, la_a1, la_a2, la_a3,
           g0_w1, g0_w2, g0_w3, g1_w1, g1_w2, g1_w3):
    b, seq = inputs.shape
    n_in = b * seq
    n_first = b * seq * SAMPLE_NUM
    n_abc = 2 * n_in + n_first
    n_pair = 2 * n_first

    ff_t = jnp.transpose(first_adj, (2, 0, 1)).reshape(-1).astype(jnp.int32)
    idx_abc = jnp.concatenate([
        inputs.reshape(-1).astype(jnp.int32),
        item.reshape(-1).astype(jnp.int32),
        ff_t,
    ])
    abc_row = jnp.right_shift(idx_abc, 1)
    abc_par = jnp.bitwise_and(idx_abc, 1).astype(jnp.float32)
    pair_idx = jnp.concatenate([ff_t * 2, ff_t * 2 + 1])
    adj_flat = adj_all.reshape(-1).astype(jnp.int32)

    emb2 = emb.reshape(-1, 2 * DIM)
    out1, out2, out2i = _sc_gather(emb2, abc_row, pair_idx, adj_flat,
                                   n_abc=n_abc, n_pair=n_pair)

    h_rows = out1[:n_in].reshape(b, seq, 2 * DIM)
    item_rows = out1[n_in:2 * n_in].reshape(b, seq, 2 * DIM)
    first_rows = out1[2 * n_in:].reshape(SAMPLE_NUM, b * seq, 2 * DIM)
    second_rows = out2.reshape(2, SAMPLE_NUM, b * seq, 2 * DIM)

    h_par = abc_par[:n_in].reshape(b, seq)
    item_par = abc_par[n_in:2 * n_in].reshape(b, seq)
    first_par = abc_par[2 * n_in:].reshape(SAMPLE_NUM, b * seq)
    second_par = jnp.bitwise_and(out2i, 1).astype(jnp.float32).reshape(
        2, SAMPLE_NUM, b * seq)

    amat = jnp.concatenate([la_a0, la_a1, la_a2, la_a3], axis=1)

    out = _tc_compute(emb2, adj.astype(jnp.int32), mask_item, amat,
                      g0_w1, g0_w2, g0_w3, g1_w1, g1_w2, g1_w3,
                      h_rows, h_par, item_rows, item_par,
                      first_rows, first_par, second_rows, second_par, bb=32)
    return out


# speedup vs baseline: 1.6435x; 1.0551x over previous
import functools

import jax
import jax.numpy as jnp
from jax import lax
from jax.experimental import pallas as pl
from jax.experimental.pallas import tpu as pltpu
from jax.experimental.pallas import tpu_sc as plsc

DIM = 64
L = 20
SAMPLE_NUM = 12
ALPHA = 0.2


def _sc_gather(emb, abc_idx, pair_idx, adj_flat, n_abc, n_pair):
    info = plsc.get_sparse_core_info()
    NC, NS = info.num_cores, info.num_subcores
    NW = NC * NS

    per_a = n_abc // NW
    per_p = n_pair // NW
    C = 128
    n_ca = per_a // C
    n_cp = per_p // C

    mesh = plsc.VectorSubcoreMesh(core_axis_name="c", subcore_axis_name="s")

    @functools.partial(
        pl.kernel,
        mesh=mesh,
        compiler_params=pltpu.CompilerParams(use_tc_tiling_on_sc=False),
        out_type=[
            jax.ShapeDtypeStruct((n_abc, DIM), jnp.float32),
            jax.ShapeDtypeStruct((n_pair, DIM), jnp.float32),
        ],
        scratch_types=[
            pltpu.VMEM((max(per_a, per_p),), jnp.int32),
            pltpu.VMEM((C,), jnp.int32),
            pltpu.VMEM((C, DIM), jnp.float32),
            pltpu.SemaphoreType.DMA,
        ],
    )
    def k(emb_hbm, abc_hbm, pair_hbm, adjflat_hbm,
          out1_hbm, out2_hbm,
          ibuf, pbuf, rows, sem):
        wid = lax.axis_index("s") * NC + lax.axis_index("c")

        base_a = wid * per_a
        pltpu.sync_copy(abc_hbm.at[pl.ds(base_a, per_a)],
                        ibuf.at[pl.ds(0, per_a)])

        def pa_body(c, carry):
            pltpu.async_copy(
                emb_hbm.at[ibuf.at[pl.ds(c * C, C)]], rows, sem).wait()
            pltpu.sync_copy(rows, out1_hbm.at[pl.ds(base_a + c * C, C)])
            return carry

        lax.fori_loop(0, n_ca, pa_body, 0, unroll=False)

        base_p = wid * per_p
        pltpu.sync_copy(pair_hbm.at[pl.ds(base_p, per_p)], ibuf)

        def pb_body(c, carry):
            pltpu.async_copy(
                adjflat_hbm.at[ibuf.at[pl.ds(c * C, C)]], pbuf, sem).wait()
            pltpu.async_copy(emb_hbm.at[pbuf], rows, sem).wait()
            pltpu.sync_copy(rows, out2_hbm.at[pl.ds(base_p + c * C, C)])
            return carry

        lax.fori_loop(0, n_cp, pb_body, 0, unroll=False)

    return k(emb, abc_idx, pair_idx, adj_flat)


def _leaky(x):
    return jnp.where(x >= 0, x, ALPHA * x)


def _bmm(x, y, dims):
    return lax.dot_general(x, y, dims, preferred_element_type=jnp.float32)


def _gagg(selfv, nb, s_m, w1, w2, w3):
    kk, m, _ = nb.shape
    x = nb * s_m[None]
    a = _leaky(_bmm(x.reshape(kk * m, DIM), w1, (((1,), (0,)), ((), ()))))
    logits = _bmm(a, w2, (((1,), (0,)), ((), ()))).reshape(kk, m)
    mx = jnp.max(logits, axis=0, keepdims=True)
    exl = jnp.exp(logits - mx)
    attn = exl / jnp.sum(exl, axis=0, keepdims=True)
    agg = jnp.sum(attn[..., None] * nb, axis=0)
    cat = jnp.concatenate([selfv, agg], axis=-1)
    o = _bmm(cat, w3, (((1,), (0,)), ((), ())))
    return jnp.maximum(o, 0.0)


def _tc_body(adj_ref, mask_ref, a_ref, g0w1_ref, g0w2_ref,
             g0w3_ref, g1w1_ref, g1w2_ref, g1w3_ref, h_ref,
             item_ref, first_ref, second_ref,
             out_ref):
    bb = h_ref.shape[0]
    m = bb * L
    h = h_ref[...]
    amat = a_ref[...]

    adj = adj_ref[...]
    e = []
    for kk in range(4):
        ha = h * amat[:, kk][None, None, :]
        ek = _bmm(ha, h, (((2,), (2,)), ((0,), (0,))))
        e.append(_leaky(ek))
    alpha = jnp.full_like(e[0], -9e15)
    alpha = jnp.where(adj == 1, e[0], alpha)
    alpha = jnp.where(adj == 2, e[1], alpha)
    alpha = jnp.where(adj == 3, e[2], alpha)
    alpha = jnp.where(adj == 4, e[3], alpha)
    mxl = jnp.max(alpha, axis=-1, keepdims=True)
    ex = jnp.exp(alpha - mxl)
    alpha = ex / jnp.sum(ex, axis=-1, keepdims=True)
    h_local = _bmm(alpha, h, (((2,), (1,)), ((0,), (0,))))

    mask = mask_ref[...]
    item_emb = item_ref[...] * mask[..., None]
    s = jnp.sum(item_emb, axis=1) / jnp.sum(mask, axis=-1)[..., None]
    s_bl = jnp.broadcast_to(s[:, None, :], (bb, L, DIM)).reshape(m, DIM)
    s_kbl = jnp.broadcast_to(s_bl[None], (SAMPLE_NUM, m, DIM)).reshape(
        SAMPLE_NUM * m, DIM)

    first = first_ref[...]
    second = second_ref[...]

    g0w1, g0w2, g0w3 = g0w1_ref[...], g0w2_ref[...], g0w3_ref[...]
    g1w1, g1w2, g1w3 = g1w1_ref[...], g1w2_ref[...], g1w3_ref[...]

    h_flat = h.reshape(m, DIM)
    first_flat = first.reshape(SAMPLE_NUM * m, DIM)
    hop0a = _gagg(h_flat, first, s_bl, g0w1, g0w2, g0w3)
    hop0b = _gagg(first_flat, second.reshape(2, SAMPLE_NUM * m, DIM),
                  s_kbl, g0w1, g0w2, g0w3)
    hop1 = _gagg(hop0a, hop0b.reshape(SAMPLE_NUM, m, DIM),
                 s_bl, g1w1, g1w2, g1w3)

    out_ref[...] = h_local + hop1.reshape(bb, L, DIM)


def _tc_compute(adj, mask_item, amat, g0w1, g0w2, g0w3, g1w1, g1w2,
                g1w3, h_rows, item_rows, first_rows, second_rows, bb):
    b = adj.shape[0]
    grid = (b // bb,)
    m = bb * L

    def blk(*shape):
        return pl.BlockSpec((bb,) + shape, lambda i: (i,) + (0,) * len(shape))

    def kblk(*lead):
        nl = len(lead)
        return pl.BlockSpec(lead + (m, DIM), lambda i: (0,) * nl + (i, 0))

    def full(arr):
        return pl.BlockSpec(arr.shape, lambda i: (0,) * arr.ndim)

    return pl.pallas_call(
        _tc_body,
        grid=grid,
        in_specs=[
            blk(L, L),
            blk(L),
            full(amat), full(g0w1), full(g0w2), full(g0w3),
            full(g1w1), full(g1w2), full(g1w3),
            blk(L, DIM),
            blk(L, DIM),
            kblk(SAMPLE_NUM),
            kblk(2, SAMPLE_NUM),
        ],
        out_specs=blk(L, DIM),
        out_shape=jax.ShapeDtypeStruct((b, L, DIM), jnp.float32),
    )(adj, mask_item, amat, g0w1, g0w2, g0w3, g1w1, g1w2, g1w3,
      h_rows, item_rows, first_rows, second_rows)


def kernel(inputs, adj, mask_item, item, first_adj, adj_all, emb,
           la_a0, la_a1, la_a2, la_a3,
           g0_w1, g0_w2, g0_w3, g1_w1, g1_w2, g1_w3):
    b, seq = inputs.shape
    n_in = b * seq
    n_first = b * seq * SAMPLE_NUM
    n_abc = 2 * n_in + n_first
    n_pair = 2 * n_first

    ff_t = jnp.transpose(first_adj, (2, 0, 1)).reshape(-1).astype(jnp.int32)
    idx_abc = jnp.concatenate([
        inputs.reshape(-1).astype(jnp.int32),
        item.reshape(-1).astype(jnp.int32),
        ff_t,
    ])
    pair_idx = jnp.concatenate([ff_t * 2, ff_t * 2 + 1])
    adj_flat = adj_all.reshape(-1).astype(jnp.int32)

    out1, out2 = _sc_gather(emb, idx_abc, pair_idx, adj_flat,
                            n_abc=n_abc, n_pair=n_pair)

    h_rows = out1[:n_in].reshape(b, seq, DIM)
    item_rows = out1[n_in:2 * n_in].reshape(b, seq, DIM)
    first_rows = out1[2 * n_in:].reshape(SAMPLE_NUM, b * seq, DIM)
    second_rows = out2.reshape(2, SAMPLE_NUM, b * seq, DIM)

    amat = jnp.concatenate([la_a0, la_a1, la_a2, la_a3], axis=1)

    out = _tc_compute(adj.astype(jnp.int32), mask_item, amat,
                      g0_w1, g0_w2, g0_w3, g1_w1, g1_w2, g1_w3,
                      h_rows, item_rows, first_rows, second_rows, bb=32)
    return out


# speedup vs baseline: 1.7064x; 1.0383x over previous
import functools

import jax
import jax.numpy as jnp
from jax import lax
from jax.experimental import pallas as pl
from jax.experimental.pallas import tpu as pltpu
from jax.experimental.pallas import tpu_sc as plsc

DIM = 64
L = 20
SAMPLE_NUM = 12
ALPHA = 0.2


def _sc_gather(emb, inp_idx, item_idx, first_idx, pair_idx, adj_flat,
               n_in, n_first, n_pair):
    info = plsc.get_sparse_core_info()
    NC, NS = info.num_cores, info.num_subcores
    NW = NC * NS

    per_i = n_in // NW
    per_f = n_first // NW
    per_p = n_pair // NW
    C = 128
    n_ci = per_i // C
    n_cf = per_f // C
    n_cp = per_p // C

    mesh = plsc.VectorSubcoreMesh(core_axis_name="c", subcore_axis_name="s")

    @functools.partial(
        pl.kernel,
        mesh=mesh,
        compiler_params=pltpu.CompilerParams(use_tc_tiling_on_sc=False),
        out_type=[
            jax.ShapeDtypeStruct((n_in, DIM), jnp.float32),
            jax.ShapeDtypeStruct((n_in, DIM), jnp.float32),
            jax.ShapeDtypeStruct((n_first, DIM), jnp.float32),
            jax.ShapeDtypeStruct((n_pair, DIM), jnp.float32),
        ],
        scratch_types=[
            pltpu.VMEM((max(per_f, per_p),), jnp.int32),
            pltpu.VMEM((C,), jnp.int32),
            pltpu.VMEM((C, DIM), jnp.float32),
            pltpu.SemaphoreType.DMA,
        ],
    )
    def k(emb_hbm, inp_hbm, item_hbm, first_hbm, pair_hbm, adjflat_hbm,
          outh_hbm, outi_hbm, outf_hbm, out2_hbm,
          ibuf, pbuf, rows, sem):
        wid = lax.axis_index("s") * NC + lax.axis_index("c")

        def gather_range(idx_hbm, out_hbm, base, n_chunks):
            pltpu.sync_copy(idx_hbm.at[pl.ds(base, n_chunks * C)],
                            ibuf.at[pl.ds(0, n_chunks * C)])

            def body(c, carry):
                pltpu.async_copy(
                    emb_hbm.at[ibuf.at[pl.ds(c * C, C)]], rows, sem).wait()
                pltpu.sync_copy(rows, out_hbm.at[pl.ds(base + c * C, C)])
                return carry

            lax.fori_loop(0, n_chunks, body, 0, unroll=False)

        gather_range(inp_hbm, outh_hbm, wid * per_i, n_ci)
        gather_range(item_hbm, outi_hbm, wid * per_i, n_ci)
        gather_range(first_hbm, outf_hbm, wid * per_f, n_cf)

        base_p = wid * per_p
        pltpu.sync_copy(pair_hbm.at[pl.ds(base_p, per_p)], ibuf)

        def pb_body(c, carry):
            pltpu.async_copy(
                adjflat_hbm.at[ibuf.at[pl.ds(c * C, C)]], pbuf, sem).wait()
            pltpu.async_copy(emb_hbm.at[pbuf], rows, sem).wait()
            pltpu.sync_copy(rows, out2_hbm.at[pl.ds(base_p + c * C, C)])
            return carry

        lax.fori_loop(0, n_cp, pb_body, 0, unroll=False)

    return k(emb, inp_idx, item_idx, first_idx, pair_idx, adj_flat)


def _leaky(x):
    return jnp.where(x >= 0, x, ALPHA * x)


def _bmm(x, y, dims):
    return lax.dot_general(x, y, dims, preferred_element_type=jnp.float32)


def _gagg(selfv, nb, s_m, w1, w2, w3):
    kk, m, _ = nb.shape
    x = nb * s_m[None]
    a = _leaky(_bmm(x.reshape(kk * m, DIM), w1, (((1,), (0,)), ((), ()))))
    logits = _bmm(a, w2, (((1,), (0,)), ((), ()))).reshape(kk, m)
    mx = jnp.max(logits, axis=0, keepdims=True)
    exl = jnp.exp(logits - mx)
    attn = exl / jnp.sum(exl, axis=0, keepdims=True)
    agg = jnp.sum(attn[..., None] * nb, axis=0)
    cat = jnp.concatenate([selfv, agg], axis=-1)
    o = _bmm(cat, w3, (((1,), (0,)), ((), ())))
    return jnp.maximum(o, 0.0)


def _tc_body(adj_ref, mask_ref, a_ref, g0w1_ref, g0w2_ref,
             g0w3_ref, g1w1_ref, g1w2_ref, g1w3_ref, h_ref,
             item_ref, first_ref, second_ref,
             out_ref):
    bb = h_ref.shape[0]
    m = bb * L
    h = h_ref[...]
    amat = a_ref[...]

    adj = adj_ref[...]
    e = []
    for kk in range(4):
        ha = h * amat[:, kk][None, None, :]
        ek = _bmm(ha, h, (((2,), (2,)), ((0,), (0,))))
        e.append(_leaky(ek))
    alpha = jnp.full_like(e[0], -9e15)
    alpha = jnp.where(adj == 1, e[0], alpha)
    alpha = jnp.where(adj == 2, e[1], alpha)
    alpha = jnp.where(adj == 3, e[2], alpha)
    alpha = jnp.where(adj == 4, e[3], alpha)
    mxl = jnp.max(alpha, axis=-1, keepdims=True)
    ex = jnp.exp(alpha - mxl)
    alpha = ex / jnp.sum(ex, axis=-1, keepdims=True)
    h_local = _bmm(alpha, h, (((2,), (1,)), ((0,), (0,))))

    mask = mask_ref[...]
    item_emb = item_ref[...] * mask[..., None]
    s = jnp.sum(item_emb, axis=1) / jnp.sum(mask, axis=-1)[..., None]
    s_bl = jnp.broadcast_to(s[:, None, :], (bb, L, DIM)).reshape(m, DIM)
    s_kbl = jnp.broadcast_to(s_bl[None], (SAMPLE_NUM, m, DIM)).reshape(
        SAMPLE_NUM * m, DIM)

    first = first_ref[...]
    second = second_ref[...]

    g0w1, g0w2, g0w3 = g0w1_ref[...], g0w2_ref[...], g0w3_ref[...]
    g1w1, g1w2, g1w3 = g1w1_ref[...], g1w2_ref[...], g1w3_ref[...]

    h_flat = h.reshape(m, DIM)
    first_flat = first.reshape(SAMPLE_NUM * m, DIM)
    hop0a = _gagg(h_flat, first, s_bl, g0w1, g0w2, g0w3)
    hop0b = _gagg(first_flat, second.reshape(2, SAMPLE_NUM * m, DIM),
                  s_kbl, g0w1, g0w2, g0w3)
    hop1 = _gagg(hop0a, hop0b.reshape(SAMPLE_NUM, m, DIM),
                 s_bl, g1w1, g1w2, g1w3)

    out_ref[...] = h_local + hop1.reshape(bb, L, DIM)


def _tc_compute(adj, mask_item, amat, g0w1, g0w2, g0w3, g1w1, g1w2,
                g1w3, h_rows, item_rows, first_rows, second_rows, bb):
    b = adj.shape[0]
    grid = (b // bb,)
    m = bb * L

    def blk(*shape):
        return pl.BlockSpec((bb,) + shape, lambda i: (i,) + (0,) * len(shape))

    def kblk(*lead):
        nl = len(lead)
        return pl.BlockSpec(lead + (m, DIM), lambda i: (0,) * nl + (i, 0))

    def full(arr):
        return pl.BlockSpec(arr.shape, lambda i: (0,) * arr.ndim)

    return pl.pallas_call(
        _tc_body,
        grid=grid,
        in_specs=[
            blk(L, L),
            blk(L),
            full(amat), full(g0w1), full(g0w2), full(g0w3),
            full(g1w1), full(g1w2), full(g1w3),
            blk(L, DIM),
            blk(L, DIM),
            kblk(SAMPLE_NUM),
            kblk(2, SAMPLE_NUM),
        ],
        out_specs=blk(L, DIM),
        out_shape=jax.ShapeDtypeStruct((b, L, DIM), jnp.float32),
    )(adj, mask_item, amat, g0w1, g0w2, g0w3, g1w1, g1w2, g1w3,
      h_rows, item_rows, first_rows, second_rows)


def kernel(inputs, adj, mask_item, item, first_adj, adj_all, emb,
           la_a0, la_a1, la_a2, la_a3,
           g0_w1, g0_w2, g0_w3, g1_w1, g1_w2, g1_w3):
    b, seq = inputs.shape
    n_in = b * seq
    n_first = b * seq * SAMPLE_NUM
    n_abc = 2 * n_in + n_first
    n_pair = 2 * n_first

    ff_t = jnp.transpose(
        first_adj.reshape(n_in, SAMPLE_NUM).astype(jnp.int32)).reshape(-1)
    pair_idx = jnp.concatenate([ff_t * 2, ff_t * 2 + 1])
    adj_flat = adj_all.reshape(-1).astype(jnp.int32)

    out_h, out_item, out_first, out2 = _sc_gather(
        emb, inputs.reshape(-1).astype(jnp.int32),
        item.reshape(-1).astype(jnp.int32), ff_t, pair_idx, adj_flat,
        n_in=n_in, n_first=n_first, n_pair=n_pair)

    h_rows = out_h.reshape(b, seq, DIM)
    item_rows = out_item.reshape(b, seq, DIM)
    first_rows = out_first.reshape(SAMPLE_NUM, b * seq, DIM)
    second_rows = out2.reshape(2, SAMPLE_NUM, b * seq, DIM)

    amat = jnp.concatenate([la_a0, la_a1, la_a2, la_a3], axis=1)

    out = _tc_compute(adj.astype(jnp.int32), mask_item, amat,
                      g0_w1, g0_w2, g0_w3, g1_w1, g1_w2, g1_w3,
                      h_rows, item_rows, first_rows, second_rows, bb=32)
    return out
